# baseline (speedup 1.0000x reference)
import functools

import jax
import jax.numpy as jnp
from jax import lax
from jax.experimental import pallas as pl
from jax.experimental.pallas import tpu as pltpu
from jax.experimental.pallas import tpu_sc as plsc

B, N, K, D, M, CK, HID = 1, 16384, 16, 128, 2, 64, 128
E = N * K

_NSPLIT = 4
_EH = E // _NSPLIT
_NC, _NS = 2, 16
_NW = _NC * _NS
_PER_W = _EH // _NW
_CHUNK = 256
_NCHUNK = _PER_W // _CHUNK
_NPAIR = _NCHUNK // 2


def _sc_gather_body(xidx, yidx, table, xout, yout,
                    idx0, idx1, rows0, rows1, gsem0, gsem1, ssem0, ssem1):
    wid = lax.axis_index("s") * _NC + lax.axis_index("c")
    base = wid * _PER_W

    def run(src_idx, dst):
        def fire(i, idx_v, rows_v, gsem):
            pltpu.sync_copy(src_idx.at[pl.ds(base + i * _CHUNK, _CHUNK)], idx_v)
            pltpu.async_copy(table.at[idx_v], rows_v, gsem)

        def gwait(idx_v, rows_v, gsem):
            pltpu.make_async_copy(table.at[idx_v], rows_v, gsem).wait()

        def sfire(i, rows_v, ssem):
            pltpu.async_copy(rows_v, dst.at[pl.ds(base + i * _CHUNK, _CHUNK)], ssem)

        def swait(rows_v, ssem):
            pltpu.make_async_copy(rows_v, dst.at[pl.ds(base, _CHUNK)], ssem).wait()

        fire(0, idx0, rows0, gsem0)

        def pair(j, _):
            i0 = 2 * j
            gwait(idx0, rows0, gsem0)
            fire(i0 + 1, idx1, rows1, gsem1)
            sfire(i0, rows0, ssem0)
            gwait(idx1, rows1, gsem1)

            @pl.when(j < _NPAIR - 1)
            def _():
                swait(rows0, ssem0)
                fire(i0 + 2, idx0, rows0, gsem0)

            sfire(i0 + 1, rows1, ssem1)

            @pl.when(j < _NPAIR - 1)
            def _():
                swait(rows1, ssem1)

            return 0

        lax.fori_loop(0, _NPAIR, pair, 0)
        swait(rows0, ssem0)
        swait(rows1, ssem1)

    run(xidx, xout)
    run(yidx, yout)


@functools.cache
def _sc_gather():
    return pl.kernel(
        _sc_gather_body,
        out_type=[
            jax.ShapeDtypeStruct((_EH, D), jnp.float32),
            jax.ShapeDtypeStruct((_EH, D), jnp.float32),
        ],
        mesh=plsc.VectorSubcoreMesh(core_axis_name="c", subcore_axis_name="s"),
        scratch_types=[
            pltpu.VMEM((_CHUNK,), jnp.int32),
            pltpu.VMEM((_CHUNK,), jnp.int32),
            pltpu.VMEM((_CHUNK, D), jnp.float32),
            pltpu.VMEM((_CHUNK, D), jnp.float32),
            pltpu.SemaphoreType.DMA,
            pltpu.SemaphoreType.DMA,
            pltpu.SemaphoreType.DMA,
            pltpu.SemaphoreType.DMA,
        ],
    )


_BLK = 4096
_GRID = _EH // _BLK


def _gelu_pre(x):
    return x * (1.0 + lax.erf(x))


def _mlp_body(xg, yg, ga0, ga1, w1x, w1y, b1, w2a, w2b, b2, w3a, w3b, b3s, out):
    h = jnp.dot(xg[...], w1x[...], preferred_element_type=jnp.float32)
    h = h + jnp.dot(yg[...], w1y[...], preferred_element_type=jnp.float32)
    h = _gelu_pre(h + b1[...])
    ha = jnp.dot(h[:, :HID], w2a[...], preferred_element_type=jnp.float32)
    hb = jnp.dot(h[:, HID:], w2b[...], preferred_element_type=jnp.float32)
    b2v = b2[...]
    ha = _gelu_pre(ha + b2v[:, :HID])
    hb = _gelu_pre(hb + b2v[:, HID:])
    bf = jnp.dot(ha, w3a[...], preferred_element_type=jnp.float32)
    bf = bf + jnp.dot(hb, w3b[...], preferred_element_type=jnp.float32)
    b3 = bf.reshape(_BLK // 128, 128, 128)
    r = lax.broadcasted_iota(jnp.int32, (128, 128), 0)
    c = lax.broadcasted_iota(jnp.int32, (128, 128), 1)
    eye = jnp.where(r == c, 1.0, 0.0)
    bias = jnp.sum(b3 * eye[None], axis=1)
    out[...] = ga0[...] + ga1[...] - bias - b3s[0]


def _full(shape):
    return pl.BlockSpec(shape, lambda e: tuple(0 for _ in shape))


_mlp_call = pl.pallas_call(
    _mlp_body,
    grid=(_GRID,),
    in_specs=[
        pl.BlockSpec((_BLK, D), lambda e: (e, 0)),
        pl.BlockSpec((_BLK, D), lambda e: (e, 0)),
        pl.BlockSpec((_BLK // 128, 128), lambda e: (e, 0)),
        pl.BlockSpec((_BLK // 128, 128), lambda e: (e, 0)),
        _full((D, 2 * HID)),
        _full((D, 2 * HID)),
        _full((1, 2 * HID)),
        _full((HID, HID)),
        _full((HID, HID)),
        _full((1, 2 * HID)),
        _full((HID, 128)),
        _full((HID, 128)),
        pl.BlockSpec(memory_space=pltpu.SMEM),
    ],
    out_specs=pl.BlockSpec((_BLK // 128, 128), lambda e: (e, 0)),
    out_shape=jax.ShapeDtypeStruct((_EH // 128, 128), jnp.float32),
    compiler_params=pltpu.CompilerParams(
        dimension_semantics=("arbitrary",),
    ),
)


def kernel(backbone_features, indices, gather_affinities, emb, W1, b1, W2, b2, W3, b3,
           bW1, bb1, bW2, bb2, bW3, bb3):
    table = backbone_features.reshape(N, D)
    idx = indices.astype(jnp.int32)
    x_idx = idx[0, 1].reshape(E)
    y_idx = idx[0, 2].reshape(E)

    s = 0.7071067811865476
    w1x = jnp.concatenate([bW1[0, :D, :], bW1[1, :D, :]], axis=1) * s
    w1y = jnp.concatenate([bW1[0, D:, :], bW1[1, D:, :]], axis=1) * s
    b1c = (jnp.concatenate([bb1[0], bb1[1]]) * s).reshape(1, 2 * HID)
    w2a = bW2[0] * 0.5
    w2b = bW2[1] * 0.5
    b2c = (jnp.concatenate([bb2[0], bb2[1]]) * s).reshape(1, 2 * HID)
    w3a = jnp.broadcast_to((bW3[0, :, 0] * s).reshape(HID, 1), (HID, 128))
    w3b = jnp.broadcast_to((bW3[1, :, 0] * s).reshape(HID, 1), (HID, 128))
    b3s = (bb3[0] + bb3[1]).reshape(1)

    ga = gather_affinities.reshape(M, _NSPLIT, _EH // 128, 128)
    sc = _sc_gather()
    outs = []
    for h in range(_NSPLIT):
        xg, yg = sc(lax.dynamic_slice_in_dim(x_idx, h * _EH, _EH),
                    lax.dynamic_slice_in_dim(y_idx, h * _EH, _EH), table)
        outs.append(_mlp_call(xg, yg, ga[0, h], ga[1, h],
                              w1x, w1y, b1c, w2a, w2b, b2c, w3a, w3b, b3s))
    out = jnp.concatenate(outs)
    return out.reshape(B, N, K)


# speedup vs baseline: 1.0108x; 1.0108x over previous
import functools

import jax
import jax.numpy as jnp
from jax import lax
from jax.experimental import pallas as pl
from jax.experimental.pallas import tpu as pltpu
from jax.experimental.pallas import tpu_sc as plsc

B, N, K, D, M, CK, HID = 1, 16384, 16, 128, 2, 64, 128
E = N * K

_NSPLIT = 8
_EH = E // _NSPLIT
_NC, _NS = 2, 16
_NW = _NC * _NS
_PER_W = _EH // _NW
_CHUNK = 256
_NCHUNK = _PER_W // _CHUNK
_NPAIR = _NCHUNK // 2


def _sc_gather_body(xidx, yidx, table, xout, yout,
                    idx0, idx1, rows0, rows1, gsem0, gsem1, ssem0, ssem1):
    wid = lax.axis_index("s") * _NC + lax.axis_index("c")
    base = wid * _PER_W

    def run(src_idx, dst):
        def fire(i, idx_v, rows_v, gsem):
            pltpu.sync_copy(src_idx.at[pl.ds(base + i * _CHUNK, _CHUNK)], idx_v)
            pltpu.async_copy(table.at[idx_v], rows_v, gsem)

        def gwait(idx_v, rows_v, gsem):
            pltpu.make_async_copy(table.at[idx_v], rows_v, gsem).wait()

        def sfire(i, rows_v, ssem):
            pltpu.async_copy(rows_v, dst.at[pl.ds(base + i * _CHUNK, _CHUNK)], ssem)

        def swait(rows_v, ssem):
            pltpu.make_async_copy(rows_v, dst.at[pl.ds(base, _CHUNK)], ssem).wait()

        fire(0, idx0, rows0, gsem0)

        def pair(j, _):
            i0 = 2 * j
            gwait(idx0, rows0, gsem0)
            fire(i0 + 1, idx1, rows1, gsem1)
            sfire(i0, rows0, ssem0)
            gwait(idx1, rows1, gsem1)

            @pl.when(j < _NPAIR - 1)
            def _():
                swait(rows0, ssem0)
                fire(i0 + 2, idx0, rows0, gsem0)

            sfire(i0 + 1, rows1, ssem1)

            @pl.when(j < _NPAIR - 1)
            def _():
                swait(rows1, ssem1)

            return 0

        lax.fori_loop(0, _NPAIR, pair, 0)
        swait(rows0, ssem0)
        swait(rows1, ssem1)

    run(xidx, xout)
    run(yidx, yout)


@functools.cache
def _sc_gather():
    return pl.kernel(
        _sc_gather_body,
        out_type=[
            jax.ShapeDtypeStruct((_EH, D), jnp.float32),
            jax.ShapeDtypeStruct((_EH, D), jnp.float32),
        ],
        mesh=plsc.VectorSubcoreMesh(core_axis_name="c", subcore_axis_name="s"),
        scratch_types=[
            pltpu.VMEM((_CHUNK,), jnp.int32),
            pltpu.VMEM((_CHUNK,), jnp.int32),
            pltpu.VMEM((_CHUNK, D), jnp.float32),
            pltpu.VMEM((_CHUNK, D), jnp.float32),
            pltpu.SemaphoreType.DMA,
            pltpu.SemaphoreType.DMA,
            pltpu.SemaphoreType.DMA,
            pltpu.SemaphoreType.DMA,
        ],
    )


_BLK = 4096
_GRID = _EH // _BLK


def _gelu_pre(x):
    return x * (1.0 + lax.erf(x))


def _dot(a, b):
    return jax.lax.dot_general(a, b, (((1,), (0,)), ((), ())),
                               precision=lax.Precision.DEFAULT,
                               preferred_element_type=jnp.float32)


def _mlp_body(xg, yg, ga0, ga1, w1x, w1y, b1, w2, b2, w3i, b3s, out):
    h = _dot(xg[...], w1x[...])
    h = h + _dot(yg[...], w1y[...])
    h = _gelu_pre(h + b1[...])
    h = _gelu_pre(_dot(h, w2[...]) + b2[...])
    bf = _dot(h, w3i[...])
    b3 = bf.reshape(_BLK // 128, 128, 128)
    r = lax.broadcasted_iota(jnp.int32, (128, 128), 0)
    c = lax.broadcasted_iota(jnp.int32, (128, 128), 1)
    eye = jnp.where(r == c, 1.0, 0.0)
    bias = jnp.sum(b3 * eye[None], axis=1)
    out[...] = ga0[...] + ga1[...] - bias - b3s[0]


def _full(shape):
    return pl.BlockSpec(shape, lambda e: tuple(0 for _ in shape))


_mlp_call = pl.pallas_call(
    _mlp_body,
    grid=(_GRID,),
    in_specs=[
        pl.BlockSpec((_BLK, D), lambda e: (e, 0)),
        pl.BlockSpec((_BLK, D), lambda e: (e, 0)),
        pl.BlockSpec((_BLK // 128, 128), lambda e: (e, 0)),
        pl.BlockSpec((_BLK // 128, 128), lambda e: (e, 0)),
        _full((D, 2 * HID)),
        _full((D, 2 * HID)),
        _full((1, 2 * HID)),
        _full((2 * HID, 2 * HID)),
        _full((1, 2 * HID)),
        _full((2 * HID, 128)),
        pl.BlockSpec(memory_space=pltpu.SMEM),
    ],
    out_specs=pl.BlockSpec((_BLK // 128, 128), lambda e: (e, 0)),
    out_shape=jax.ShapeDtypeStruct((_EH // 128, 128), jnp.float32),
    compiler_params=pltpu.CompilerParams(
        dimension_semantics=("arbitrary",),
    ),
)


def kernel(backbone_features, indices, gather_affinities, emb, W1, b1, W2, b2, W3, b3,
           bW1, bb1, bW2, bb2, bW3, bb3):
    table = backbone_features.reshape(N, D)
    idx = indices.astype(jnp.int32)
    x_idx = idx[0, 1].reshape(E)
    y_idx = idx[0, 2].reshape(E)

    s = 0.7071067811865476
    w1x = jnp.concatenate([bW1[0, :D, :], bW1[1, :D, :]], axis=1) * s
    w1y = jnp.concatenate([bW1[0, D:, :], bW1[1, D:, :]], axis=1) * s
    b1c = (jnp.concatenate([bb1[0], bb1[1]]) * s).reshape(1, 2 * HID)
    z = jnp.zeros((HID, HID), jnp.float32)
    w2bd = jnp.block([[bW2[0], z], [z, bW2[1]]]) * 0.5
    b2c = (jnp.concatenate([bb2[0], bb2[1]]) * s).reshape(1, 2 * HID)
    w3i = jnp.broadcast_to(
        (jnp.concatenate([bW3[0, :, 0], bW3[1, :, 0]]) * s).reshape(2 * HID, 1),
        (2 * HID, 128))
    b3s = (bb3[0] + bb3[1]).reshape(1)

    ga = gather_affinities.reshape(M, _NSPLIT, _EH // 128, 128)
    sc = _sc_gather()
    outs = []
    for h in range(_NSPLIT):
        xg, yg = sc(lax.dynamic_slice_in_dim(x_idx, h * _EH, _EH),
                    lax.dynamic_slice_in_dim(y_idx, h * _EH, _EH), table)
        outs.append(_mlp_call(xg, yg, ga[0, h], ga[1, h],
                              w1x, w1y, b1c, w2bd, b2c, w3i, b3s))
    out = jnp.concatenate(outs)
    return out.reshape(B, N, K)
